# baseline (speedup 1.0000x reference)
import functools

import jax
import jax.numpy as jnp
from jax import lax
from jax.experimental import pallas as pl
from jax.experimental.pallas import tpu as pltpu
from jax.experimental.pallas import tpu_sc as plsc

D_MODEL = 64
BATCH = 4096
SEQ = 200
B_TOTAL = BATCH * SEQ
NUM_CORES = 2
NUM_SUBCORES = 16
NW = NUM_CORES * NUM_SUBCORES
B_PER_W = B_TOTAL // NW
CHUNK = 128
N_CHUNKS = B_PER_W // CHUNK
NBUF = 8
N_GROUPS = N_CHUNKS // NBUF

_mesh = plsc.VectorSubcoreMesh(core_axis_name="c", subcore_axis_name="s")


@functools.partial(
    pl.kernel,
    mesh=_mesh,
    out_type=jax.ShapeDtypeStruct((NW, N_CHUNKS, CHUNK, D_MODEL), jnp.float32),
    scratch_types=[
        pltpu.VMEM((N_CHUNKS, CHUNK), jnp.int32),
        *([pltpu.VMEM((CHUNK, D_MODEL), jnp.float32)] * NBUF),
        *([pltpu.SemaphoreType.DMA] * NBUF),
        *([pltpu.SemaphoreType.DMA] * NBUF),
    ],
    compiler_params=pltpu.CompilerParams(use_tc_tiling_on_sc=False),
)
def _embed_sc(idx_hbm, table_hbm, out_hbm, idx_v, *bufs):
    rows = bufs[:NBUF]
    gsems = bufs[NBUF:2 * NBUF]
    wsems = bufs[2 * NBUF:3 * NBUF]
    wid = lax.axis_index("s") * NUM_CORES + lax.axis_index("c")
    pltpu.sync_copy(idx_hbm.at[wid], idx_v)

    def g_copy(j, b):
        return pltpu.make_async_copy(
            table_hbm.at[idx_v.at[j]], rows[b], gsems[b])

    def w_copy(j, b):
        return pltpu.make_async_copy(
            rows[b], out_hbm.at[wid, j], wsems[b])

    for b in range(NBUF):
        g_copy(b, b).start()

    def group_body(g, carry):
        for b in range(NBUF):
            j = g * NBUF + b
            g_copy(j, b).wait()
            w_copy(j, b).start()
            w_copy(j, b).wait()
            g_copy(j + NBUF, b).start()
        return carry

    lax.fori_loop(0, N_GROUPS - 1, group_body, 0)

    for b in range(NBUF):
        j = (N_GROUPS - 1) * NBUF + b
        g_copy(j, b).wait()
        w_copy(j, b).start()
        w_copy(j, b).wait()


def kernel(input_ids, weight):
    idx = input_ids.reshape(NW, N_CHUNKS, CHUNK)
    out = _embed_sc(idx, weight)
    return out.reshape(BATCH, SEQ, D_MODEL)


# speedup vs baseline: 1.1901x; 1.1901x over previous
import functools

import jax
import jax.numpy as jnp
from jax import lax
from jax.experimental import pallas as pl
from jax.experimental.pallas import tpu as pltpu
from jax.experimental.pallas import tpu_sc as plsc

VOCAB = 1000000
D_MODEL = 64
BATCH = 4096
SEQ = 200
NUM_CORES = 2
NUM_SUBCORES = 16
NW = NUM_CORES * NUM_SUBCORES
LANES = 128
ROWS_P = VOCAB // 2
GBUF = 4

_mesh = plsc.VectorSubcoreMesh(core_axis_name="c", subcore_axis_name="s")


def _wid():
    return lax.axis_index("s") * NUM_CORES + lax.axis_index("c")


def _skew_vecs():
    iota = lax.broadcasted_iota(jnp.int32, (16,), 0)
    pair_col = lax.shift_left(jnp.bitwise_and(iota, 1), 6)
    pair_row = lax.shift_right_logical(iota, 1)
    return iota, pair_col, pair_row


@functools.partial(
    pl.kernel,
    mesh=_mesh,
    out_type=jax.ShapeDtypeStruct((SEQ, D_MODEL, BATCH), jnp.float32),
    scratch_types=[
        pltpu.VMEM((SEQ, LANES), jnp.int32),
        pltpu.VMEM((GBUF, LANES), jnp.int32),
        pltpu.VMEM((LANES, LANES), jnp.float32),
        pltpu.VMEM((LANES, LANES), jnp.float32),
        pltpu.VMEM((LANES, LANES), jnp.float32),
        pltpu.VMEM((LANES, LANES), jnp.float32),
        pltpu.VMEM((D_MODEL, LANES), jnp.float32),
        pltpu.VMEM((D_MODEL, LANES), jnp.float32),
        pltpu.SemaphoreType.DMA,
        pltpu.SemaphoreType.DMA,
        pltpu.SemaphoreType.DMA,
        pltpu.SemaphoreType.DMA,
        pltpu.SemaphoreType.DMA,
        pltpu.SemaphoreType.DMA,
    ],
    compiler_params=pltpu.CompilerParams(needs_layout_passes=False),
)
def _gather(idst_hbm, table_hbm, out_hbm, idsb, qr,
            fet0, fet1, fet2, fet3, slab0, slab1,
            gsem0, gsem1, gsem2, gsem3, wsem0, wsem1):
    wid = _wid()
    iota, _, _ = _skew_vecs()
    fets = (fet0, fet1, fet2, fet3)
    slabs = (slab0, slab1)
    gsems = (gsem0, gsem1, gsem2, gsem3)
    wsems = (wsem0, wsem1)
    pltpu.sync_copy(idst_hbm.at[:, pl.ds(wid * LANES, LANES)], idsb)

    def make_q(t, b):
        for g in range(8):
            qr[b, pl.ds(16 * g, 16)] = lax.shift_right_logical(
                idsb[t, pl.ds(16 * g, 16)], 1)

    def g_copy(b):
        return pltpu.make_async_copy(
            table_hbm.at[qr.at[b]], fets[b], gsems[b])

    def w_copy(t, b):
        return pltpu.make_async_copy(
            slabs[b], out_hbm.at[t, :, pl.ds(wid * LANES, LANES)], wsems[b])

    def transpose_select(t, b, b2):
        offs = []
        for g in range(8):
            ids16 = idsb[t, pl.ds(16 * g, 16)]
            offs.append(lax.shift_left(jnp.bitwise_and(ids16, 1), 6))

        def kbody(k, carry):
            diag = jnp.bitwise_and(iota + k, 15)
            for g in range(8):
                base = 16 * g + iota
                for d0 in range(0, D_MODEL, 16):
                    reg = plsc.load_gather(
                        fets[b], [base, offs[g] + (d0 + diag)])
                    plsc.store_scatter(slabs[b2], [d0 + diag, base], reg)
            return carry

        lax.fori_loop(0, 16, kbody, 0)

    for b in range(GBUF):
        make_q(b, b)
        g_copy(b).start()

    def group(gi, carry):
        for b in range(GBUF):
            t = gi * GBUF + b
            b2 = b % 2
            g_copy(b).wait()

            @pl.when(t >= 2)
            def _():
                w_copy(t - 2, b2).wait()

            transpose_select(t, b, b2)
            w_copy(t, b2).start()

            @pl.when(t + GBUF < SEQ)
            def _():
                make_q(t + GBUF, b)
                g_copy(b).start()

        return carry

    lax.fori_loop(0, SEQ // GBUF, group, 0)
    for b in range(2):
        w_copy(SEQ - 2 + b, b).wait()


def kernel(input_ids, weight):
    tablep = weight.reshape(ROWS_P, 2 * D_MODEL)
    outt = _gather(input_ids.T, tablep)
    return outt.transpose(2, 0, 1)
